# Initial kernel scaffold; baseline (speedup 1.0000x reference)
import jax
import jax.numpy as jnp
from jax.experimental import pallas as pl


def kernel(x, edge_index, edge_attr, W1, b1, bn1_g, bn1_b, bn1_m, bn1_v, W2, b2, bn2_g, bn2_b, bn2_m, bn2_v):
    raise NotImplementedError("write your pallas kernel here")


# speedup vs baseline: 7.1898x; 7.1898x over previous
import functools

import jax
import jax.numpy as jnp
from jax import lax
from jax.experimental import pallas as pl
from jax.experimental.pallas import tpu as pltpu
from jax.experimental.pallas import tpu_sc as plsc

NC = 2
NS = 16
L = 16
D2 = 64
EPS = 1e-5


def _mesh():
    return plsc.VectorSubcoreMesh(
        core_axis_name="c", subcore_axis_name="s", num_cores=NC, num_subcores=NS
    )


def _make_deg_kernel(E, NP, B):
    EPT = E // (NC * NS)
    nch = EPT // B
    rpt = NP // NS

    @functools.partial(
        pl.kernel,
        mesh=_mesh(),
        compiler_params=pltpu.CompilerParams(needs_layout_passes=False, use_tc_tiling_on_sc=False),
        out_type=jax.ShapeDtypeStruct((NC * NP,), jnp.float32),
        scratch_types=[
            pltpu.VMEM((B,), jnp.int32),
            pltpu.VMEM((B,), jnp.float32),
            pltpu.VMEM_SHARED((NP,), jnp.float32),
        ],
    )
    def k(dst_hbm, ew_hbm, out_hbm, dstv, ewv, acc):
        cid = lax.axis_index("c")
        sid = lax.axis_index("s")
        wid = cid * NS + sid
        for g in range(B // L):
            ewv[pl.ds(g * L, L)] = jnp.zeros((L,), jnp.float32)
        for b in range(rpt // B):
            pltpu.sync_copy(ewv, acc.at[pl.ds(sid * rpt + b * B, B)])
        plsc.subcore_barrier()

        def body(i, carry):
            base = wid * EPT + i * B
            pltpu.sync_copy(dst_hbm.at[pl.ds(base, B)], dstv)
            pltpu.sync_copy(ew_hbm.at[pl.ds(base, B)], ewv)
            pltpu.sync_copy(ewv, acc.at[dstv], add=True)
            return carry

        lax.fori_loop(0, nch, body, 0)
        plsc.subcore_barrier()
        pltpu.sync_copy(
            acc.at[pl.ds(sid * rpt, rpt)],
            out_hbm.at[pl.ds(cid * NP + sid * rpt, rpt)],
        )

    return k


def _make_scalar_msg_kernel(E, NP, B):
    EPT = E // (NC * NS)
    nch = EPT // B
    rpt = NP // NS

    @functools.partial(
        pl.kernel,
        mesh=_mesh(),
        compiler_params=pltpu.CompilerParams(needs_layout_passes=False, use_tc_tiling_on_sc=False),
        out_type=jax.ShapeDtypeStruct((NC * NP,), jnp.float32),
        scratch_types=[
            pltpu.VMEM((B,), jnp.int32),
            pltpu.VMEM((B,), jnp.int32),
            pltpu.VMEM((B,), jnp.float32),
            pltpu.VMEM((B,), jnp.float32),
            pltpu.VMEM((NP,), jnp.float32),
            pltpu.VMEM_SHARED((NP,), jnp.float32),
        ],
    )
    def k(src_hbm, dst_hbm, ew_hbm, u_hbm, out_hbm, srcv, dstv, ewv, msgv, uv, acc):
        cid = lax.axis_index("c")
        sid = lax.axis_index("s")
        wid = cid * NS + sid
        pltpu.sync_copy(u_hbm, uv)
        for g in range(B // L):
            msgv[pl.ds(g * L, L)] = jnp.zeros((L,), jnp.float32)
        for b in range(rpt // B):
            pltpu.sync_copy(msgv, acc.at[pl.ds(sid * rpt + b * B, B)])
        plsc.subcore_barrier()

        def body(i, carry):
            base = wid * EPT + i * B
            pltpu.sync_copy(src_hbm.at[pl.ds(base, B)], srcv)
            pltpu.sync_copy(dst_hbm.at[pl.ds(base, B)], dstv)
            pltpu.sync_copy(ew_hbm.at[pl.ds(base, B)], ewv)
            for g in range(B // L):
                idx = srcv[pl.ds(g * L, L)]
                vals = plsc.load_gather(uv, [idx])
                msgv[pl.ds(g * L, L)] = vals * ewv[pl.ds(g * L, L)]
            pltpu.sync_copy(msgv, acc.at[dstv], add=True)
            return carry

        lax.fori_loop(0, nch, body, 0)
        plsc.subcore_barrier()
        pltpu.sync_copy(
            acc.at[pl.ds(sid * rpt, rpt)],
            out_hbm.at[pl.ds(cid * NP + sid * rpt, rpt)],
        )

    return k


def _make_wide_msg_kernel(E, NP, B):
    EPT = E // (NC * NS)
    nch = EPT // B
    rpt = NP // NS

    @functools.partial(
        pl.kernel,
        mesh=_mesh(),
        compiler_params=pltpu.CompilerParams(needs_layout_passes=False, use_tc_tiling_on_sc=False),
        out_type=jax.ShapeDtypeStruct((NC * NP, D2), jnp.float32),
        scratch_types=[
            pltpu.VMEM((B,), jnp.int32),
            pltpu.VMEM((B,), jnp.int32),
            pltpu.VMEM((B,), jnp.float32),
            pltpu.VMEM((B, D2), jnp.float32),
            pltpu.SemaphoreType.DMA,
            pltpu.VMEM_SHARED((NP, D2), jnp.float32),
        ],
    )
    def k(src_hbm, dst_hbm, ew_hbm, v_hbm, out_hbm, srcv, dstv, ewv, rows, sem, acc):
        cid = lax.axis_index("c")
        sid = lax.axis_index("s")
        wid = cid * NS + sid
        for r in range(B):
            for j in range(D2 // L):
                rows[r, pl.ds(j * L, L)] = jnp.zeros((L,), jnp.float32)
        for b in range(rpt // B):
            pltpu.sync_copy(rows, acc.at[pl.ds(sid * rpt + b * B, B)])
        plsc.subcore_barrier()
        iota = lax.iota(jnp.int32, L)

        def body(i, carry):
            base = wid * EPT + i * B
            pltpu.sync_copy(src_hbm.at[pl.ds(base, B)], srcv)
            pltpu.sync_copy(dst_hbm.at[pl.ds(base, B)], dstv)
            pltpu.sync_copy(ew_hbm.at[pl.ds(base, B)], ewv)
            pltpu.async_copy(v_hbm.at[srcv], rows, sem).wait()
            for g in range(B // L):
                ew16 = ewv[pl.ds(g * L, L)]
                ridx = iota + g * L
                for col in range(D2):
                    cvec = jnp.full((L,), col, jnp.int32)
                    vals = plsc.load_gather(rows, [ridx, cvec]) * ew16
                    plsc.store_scatter(rows, [ridx, cvec], vals)
            pltpu.sync_copy(rows, acc.at[dstv], add=True)
            return carry

        lax.fori_loop(0, nch, body, 0)
        plsc.subcore_barrier()
        pltpu.sync_copy(
            acc.at[pl.ds(sid * rpt, rpt)],
            out_hbm.at[pl.ds(cid * NP + sid * rpt, rpt)],
        )

    return k


def _prep_kernel(d0_ref, d1_ref, x_ref, dinv_ref, u_ref):
    deg = d0_ref[...] + d1_ref[...] + 1.0
    dinv = jnp.where(deg > 0, lax.rsqrt(deg), 0.0)
    dinv_ref[...] = dinv
    u_ref[...] = dinv * x_ref[...]


def _dense_kernel(t_ref, dinv_ref, x_ref, w1_ref, b1_ref, g1_ref, bb1_ref,
                  m1_ref, v1_ref, w2_ref, xw2_ref, vout_ref):
    inv1 = lax.rsqrt(v1_ref[...] + EPS)
    a = w1_ref[...] * g1_ref[...] * inv1
    c = (b1_ref[...] - m1_ref[...]) * inv1 * g1_ref[...] + bb1_ref[...]
    dinv = dinv_ref[...]
    s = dinv * t_ref[...] + dinv * dinv * x_ref[...]
    h1 = jnp.maximum(s * a + c, 0.0)
    xw2 = jnp.dot(h1, w2_ref[...], preferred_element_type=jnp.float32)
    xw2_ref[...] = xw2
    vout_ref[...] = dinv * xw2


def _final_kernel(a0_ref, a1_ref, dinv_ref, xw2_ref, b2_ref, g2_ref, bb2_ref,
                  m2_ref, v2_ref, out_ref):
    dinv = dinv_ref[...]
    xw2 = xw2_ref[...]
    o = dinv * (a0_ref[...] + a1_ref[...]) + dinv * dinv * xw2 + b2_ref[...]
    inv2 = lax.rsqrt(v2_ref[...] + EPS)
    out_ref[...] = jnp.maximum((o - m2_ref[...]) * inv2 * g2_ref[...] + bb2_ref[...], 0.0)


def kernel(x, edge_index, edge_attr, W1, b1, bn1_g, bn1_b, bn1_m, bn1_v,
           W2, b2, bn2_g, bn2_b, bn2_m, bn2_v):
    N = x.shape[0]
    E = edge_attr.shape[0]
    NP = 10240
    B = 80

    src = edge_index[0]
    dst = edge_index[1]
    xf = x[:, 0]
    xp = jnp.pad(xf, (0, NP - N))

    degp = _make_deg_kernel(E, NP, B)(dst, edge_attr).reshape(NC, NP)

    grid_rows = 80
    dinv_f, u_f = pl.pallas_call(
        _prep_kernel,
        out_shape=(
            jax.ShapeDtypeStruct((grid_rows, 128), jnp.float32),
            jax.ShapeDtypeStruct((grid_rows, 128), jnp.float32),
        ),
    )(degp[0].reshape(grid_rows, 128), degp[1].reshape(grid_rows, 128),
      xp.reshape(grid_rows, 128))
    dinv_f = dinv_f.reshape(NP)
    u_f = u_f.reshape(NP)

    tp = _make_scalar_msg_kernel(E, NP, B)(src, dst, edge_attr, u_f).reshape(NC, NP)
    t_f = tp[0] + tp[1]

    BLK = 1024
    col = lambda z: z.reshape(NP, 1)
    row = lambda z: z.reshape(1, -1)
    xw2, v = pl.pallas_call(
        _dense_kernel,
        grid=(NP // BLK,),
        in_specs=[
            pl.BlockSpec((BLK, 1), lambda i: (i, 0)),
            pl.BlockSpec((BLK, 1), lambda i: (i, 0)),
            pl.BlockSpec((BLK, 1), lambda i: (i, 0)),
            pl.BlockSpec((1, 256), lambda i: (0, 0)),
            pl.BlockSpec((1, 256), lambda i: (0, 0)),
            pl.BlockSpec((1, 256), lambda i: (0, 0)),
            pl.BlockSpec((1, 256), lambda i: (0, 0)),
            pl.BlockSpec((1, 256), lambda i: (0, 0)),
            pl.BlockSpec((1, 256), lambda i: (0, 0)),
            pl.BlockSpec((256, D2), lambda i: (0, 0)),
        ],
        out_specs=(
            pl.BlockSpec((BLK, D2), lambda i: (i, 0)),
            pl.BlockSpec((BLK, D2), lambda i: (i, 0)),
        ),
        out_shape=(
            jax.ShapeDtypeStruct((NP, D2), jnp.float32),
            jax.ShapeDtypeStruct((NP, D2), jnp.float32),
        ),
    )(col(t_f), col(dinv_f), col(xp), W1, row(b1), row(bn1_g), row(bn1_b),
      row(bn1_m), row(bn1_v), W2)

    aggp = _make_wide_msg_kernel(E, NP, B)(src, dst, edge_attr, v).reshape(NC, NP, D2)

    out = pl.pallas_call(
        _final_kernel,
        grid=(NP // BLK,),
        in_specs=[
            pl.BlockSpec((BLK, D2), lambda i: (i, 0)),
            pl.BlockSpec((BLK, D2), lambda i: (i, 0)),
            pl.BlockSpec((BLK, 1), lambda i: (i, 0)),
            pl.BlockSpec((BLK, D2), lambda i: (i, 0)),
            pl.BlockSpec((1, D2), lambda i: (0, 0)),
            pl.BlockSpec((1, D2), lambda i: (0, 0)),
            pl.BlockSpec((1, D2), lambda i: (0, 0)),
            pl.BlockSpec((1, D2), lambda i: (0, 0)),
            pl.BlockSpec((1, D2), lambda i: (0, 0)),
        ],
        out_specs=pl.BlockSpec((BLK, D2), lambda i: (i, 0)),
        out_shape=jax.ShapeDtypeStruct((NP, D2), jnp.float32),
    )(aggp[0], aggp[1], col(dinv_f), xw2, row(b2), row(bn2_g), row(bn2_b),
      row(bn2_m), row(bn2_v))

    return out[:N]
